# Initial kernel scaffold; baseline (speedup 1.0000x reference)
import jax
import jax.numpy as jnp
from jax.experimental import pallas as pl


def kernel(anchor_logits, aug_logits):
    raise NotImplementedError("write your pallas kernel here")


# speedup vs baseline: 1.5603x; 1.5603x over previous
import functools

import jax
import jax.numpy as jnp
from jax.experimental import pallas as pl
from jax.experimental.pallas import tpu as pltpu

N_ROWS = 16384
N_CLS = 1000
BLK = 512
GRID = N_ROWS // BLK


def _selflabel_block(anchor_ref, aug_ref, out_ref, counts_ref, ns_ref):
    i = pl.program_id(0)
    a = anchor_ref[...]
    g = aug_ref[...]
    col = jax.lax.broadcasted_iota(jnp.int32, (BLK, N_CLS), 1)

    row_max = jnp.max(a, axis=1, keepdims=True)
    t = jnp.min(jnp.where(a == row_max, col, N_CLS), axis=1, keepdims=True)

    g_max = jnp.max(g, axis=1, keepdims=True)
    s = jnp.sum(jnp.exp(g - g_max), axis=1, keepdims=True)
    lse = jnp.log(s) + g_max

    onehot = col == t
    g_t = jnp.sum(jnp.where(onehot, g, 0.0), axis=1, keepdims=True)
    nll = lse - g_t

    cnt_blk = jnp.sum(onehot.astype(jnp.float32), axis=0)[None, :]
    ns_blk = jnp.sum(jnp.where(onehot, nll, 0.0), axis=0)[None, :]

    @pl.when(i == 0)
    def _init():
        counts_ref[...] = jnp.zeros_like(counts_ref)
        ns_ref[...] = jnp.zeros_like(ns_ref)

    counts_ref[...] += cnt_blk
    ns_ref[...] += ns_blk

    @pl.when(i == GRID - 1)
    def _finalize():
        c = counts_ref[...]
        ns = ns_ref[...]
        nz = c > 0.0
        k = jnp.sum(nz.astype(jnp.float32), axis=1, keepdims=True)
        per_cls = jnp.where(nz, ns / jnp.where(nz, c, 1.0), 0.0)
        out_ref[...] = jnp.sum(per_cls, axis=1, keepdims=True) / k


@functools.partial(jax.jit, static_argnames=("interpret",))
def kernel(anchor_logits, aug_logits, interpret=False):
    out = pl.pallas_call(
        _selflabel_block,
        grid=(GRID,),
        in_specs=[
            pl.BlockSpec((BLK, N_CLS), lambda i: (i, 0)),
            pl.BlockSpec((BLK, N_CLS), lambda i: (i, 0)),
        ],
        out_specs=pl.BlockSpec((1, 1), lambda i: (0, 0)),
        out_shape=jax.ShapeDtypeStruct((1, 1), jnp.float32),
        scratch_shapes=[
            pltpu.VMEM((1, N_CLS), jnp.float32),
            pltpu.VMEM((1, N_CLS), jnp.float32),
        ],
        interpret=interpret,
    )(anchor_logits, aug_logits)
    return out[0, 0]
